# baseline (speedup 1.0000x reference)
import functools

import jax
import jax.numpy as jnp
from jax import lax
from jax.experimental import pallas as pl
from jax.experimental.pallas import tpu as pltpu
from jax.experimental.pallas import tpu_sc as plsc

CORPUS_SIZE = 100000
EMBEDDING_DIM = 100
BATCH = 1024
SEQ_LEN = 20

_INFO = plsc.get_sparse_core_info()
NC = _INFO.num_cores
NS = _INFO.num_subcores
LANES = _INFO.num_lanes
NW = NC * NS

SENT_PER_W = BATCH // NW
TOK_PER_W = SENT_PER_W * SEQ_LEN
IDX_CHUNK = 128
N_CHUNKS = TOK_PER_W // IDX_CHUNK
GROUPS = IDX_CHUNK // LANES


def _newton_sqrt(x):
    i = lax.bitcast_convert_type(x, jnp.int32)
    y = lax.bitcast_convert_type(jnp.int32(0x5F3759DF) - (i >> 1), jnp.float32)
    for _ in range(3):
        y = y * (1.5 - 0.5 * x * y * y)
    return x * y


def _make_sc_kernel():
    mesh = plsc.VectorSubcoreMesh(core_axis_name="c", subcore_axis_name="s")

    @functools.partial(
        pl.kernel,
        mesh=mesh,
        out_type=jax.ShapeDtypeStruct((BATCH,), jnp.float32),
        compiler_params=pltpu.CompilerParams(
            needs_layout_passes=False, use_tc_tiling_on_sc=False
        ),
        scratch_types=[
            pltpu.VMEM((N_CHUNKS, IDX_CHUNK), jnp.int32),
            pltpu.VMEM((TOK_PER_W, EMBEDDING_DIM), jnp.float32),
            pltpu.VMEM((EMBEDDING_DIM,), jnp.float32),
            pltpu.VMEM((EMBEDDING_DIM,), jnp.float32),
            pltpu.VMEM((TOK_PER_W,), jnp.float32),
            pltpu.VMEM((TOK_PER_W,), jnp.float32),
            pltpu.VMEM((SENT_PER_W,), jnp.float32),
            pltpu.SemaphoreType.DMA,
        ],
    )
    def k(table_hbm, idx_hbm, u_hbm, w_hbm, out_hbm,
          idx_v, rows_v, u_v, w_v, alpha_v, num_v, res_v, sem):
        wid = lax.axis_index("s") * NC + lax.axis_index("c")

        for c in range(N_CHUNKS):
            pltpu.sync_copy(
                idx_hbm.at[pl.ds(wid * TOK_PER_W + c * IDX_CHUNK, IDX_CHUNK)],
                idx_v.at[c],
            )
        pltpu.sync_copy(u_hbm, u_v)
        pltpu.sync_copy(w_hbm, w_v)

        copies = []
        for c in range(N_CHUNKS):
            copies.append(
                pltpu.async_copy(
                    table_hbm.at[idx_v.at[c]],
                    rows_v.at[pl.ds(c * IDX_CHUNK, IDX_CHUNK)],
                    sem,
                )
            )
        for cp in copies:
            cp.wait()

        lane_iota = lax.iota(jnp.int32, LANES)

        for c in range(N_CHUNKS):
            row_bases = [
                jnp.full((LANES,), c * IDX_CHUNK + g * LANES, jnp.int32) + lane_iota
                for g in range(GROUPS)
            ]

            def dim_body(d, carry, _row_bases=row_bases):
                sqs, dots = carry
                dvec = jnp.full((LANES,), d, jnp.int32)
                u_d = plsc.load_gather(u_v, [dvec])
                w_d = plsc.load_gather(w_v, [dvec])
                new_sqs, new_dots = [], []
                for g in range(GROUPS):
                    x = plsc.load_gather(rows_v, [_row_bases[g], dvec])
                    diff = u_d - x
                    new_sqs.append(sqs[g] + diff * diff)
                    new_dots.append(dots[g] + w_d * x)
                return tuple(new_sqs), tuple(new_dots)

            zero = jnp.zeros((LANES,), jnp.float32)
            init = (tuple(zero for _ in range(GROUPS)),
                    tuple(zero for _ in range(GROUPS)))
            sqs, dots = lax.fori_loop(0, EMBEDDING_DIM, dim_body, init)

            for g in range(GROUPS):
                sq = jnp.maximum(sqs[g], 1e-12)
                dist = _newton_sqrt(sq)
                a = jnp.exp(dist)
                base = c * IDX_CHUNK + g * LANES
                alpha_v[pl.ds(base, LANES)] = a
                num_v[pl.ds(base, LANES)] = a * dots[g]

        for half in range(SENT_PER_W // LANES):
            sent = jnp.full((LANES,), half * LANES, jnp.int32) + lane_iota
            acc_a = jnp.zeros((LANES,), jnp.float32)
            acc_n = jnp.zeros((LANES,), jnp.float32)
            for j in range(SEQ_LEN):
                tok = sent * SEQ_LEN + j
                acc_a = acc_a + plsc.load_gather(alpha_v, [tok])
                acc_n = acc_n + plsc.load_gather(num_v, [tok])
            res_v[pl.ds(half * LANES, LANES)] = acc_n / acc_a

        pltpu.sync_copy(res_v, out_hbm.at[pl.ds(wid * SENT_PER_W, SENT_PER_W)])

    return k


_sc_kernel = _make_sc_kernel()


def kernel(batch_word_idxs, word_embeddings, weights, attend_u):
    idx = batch_word_idxs.astype(jnp.int32).reshape(NW * TOK_PER_W)
    w_flat = weights.reshape(EMBEDDING_DIM).astype(jnp.float32)
    out = _sc_kernel(word_embeddings, idx, attend_u, w_flat)
    return out.reshape(BATCH, 1)


# speedup vs baseline: 1.1306x; 1.1306x over previous
import functools

import jax
import jax.numpy as jnp
from jax import lax
from jax.experimental import pallas as pl
from jax.experimental.pallas import tpu as pltpu
from jax.experimental.pallas import tpu_sc as plsc

CORPUS_SIZE = 100000
EMBEDDING_DIM = 100
BATCH = 1024
SEQ_LEN = 20

_INFO = plsc.get_sparse_core_info()
NC = _INFO.num_cores
NS = _INFO.num_subcores
LANES = _INFO.num_lanes
NW = NC * NS

PADDED_DIM = 128
SENT_PER_W = BATCH // NW
TOK_PER_W = SENT_PER_W * SEQ_LEN
IDX_CHUNK = 128
N_CHUNKS = TOK_PER_W // IDX_CHUNK
GROUPS = IDX_CHUNK // LANES


def _newton_sqrt(x):
    i = lax.bitcast_convert_type(x, jnp.int32)
    y = lax.bitcast_convert_type(jnp.int32(0x5F3759DF) - (i >> 1), jnp.float32)
    for _ in range(3):
        y = y * (1.5 - 0.5 * x * y * y)
    return x * y


def _make_sc_kernel():
    mesh = plsc.VectorSubcoreMesh(core_axis_name="c", subcore_axis_name="s")

    @functools.partial(
        pl.kernel,
        mesh=mesh,
        out_type=jax.ShapeDtypeStruct((BATCH,), jnp.float32),
        compiler_params=pltpu.CompilerParams(
            needs_layout_passes=False, use_tc_tiling_on_sc=False
        ),
        scratch_types=[
            pltpu.VMEM((N_CHUNKS, IDX_CHUNK), jnp.int32),
            pltpu.VMEM((TOK_PER_W, PADDED_DIM), jnp.float32),
            pltpu.VMEM((EMBEDDING_DIM,), jnp.float32),
            pltpu.VMEM((EMBEDDING_DIM,), jnp.float32),
            pltpu.VMEM((TOK_PER_W,), jnp.float32),
            pltpu.VMEM((TOK_PER_W,), jnp.float32),
            pltpu.VMEM((SENT_PER_W,), jnp.float32),
            pltpu.SemaphoreType.DMA,
        ],
    )
    def k(table_hbm, idx_hbm, u_hbm, w_hbm, out_hbm,
          idx_v, rows_v, u_v, w_v, alpha_v, num_v, res_v, sem):
        wid = lax.axis_index("s") * NC + lax.axis_index("c")

        for c in range(N_CHUNKS):
            pltpu.sync_copy(
                idx_hbm.at[pl.ds(wid * TOK_PER_W + c * IDX_CHUNK, IDX_CHUNK)],
                idx_v.at[c],
            )
        pltpu.sync_copy(u_hbm, u_v)
        pltpu.sync_copy(w_hbm, w_v)

        copies = []
        for c in range(N_CHUNKS):
            copies.append(
                pltpu.async_copy(
                    table_hbm.at[idx_v.at[c]],
                    rows_v.at[pl.ds(c * IDX_CHUNK, IDX_CHUNK)],
                    sem,
                )
            )
        for cp in copies:
            cp.wait()

        lane_iota = lax.iota(jnp.int32, LANES)

        for c in range(N_CHUNKS):
            row_bases = [
                jnp.full((LANES,), c * IDX_CHUNK + g * LANES, jnp.int32) + lane_iota
                for g in range(GROUPS)
            ]

            def dim_body(d, carry, _row_bases=row_bases):
                sqs, dots = carry
                dvec = jnp.full((LANES,), d, jnp.int32)
                u_d = plsc.load_gather(u_v, [dvec])
                w_d = plsc.load_gather(w_v, [dvec])
                new_sqs, new_dots = [], []
                for g in range(GROUPS):
                    x = plsc.load_gather(rows_v, [_row_bases[g], dvec])
                    diff = u_d - x
                    new_sqs.append(sqs[g] + diff * diff)
                    new_dots.append(dots[g] + w_d * x)
                return tuple(new_sqs), tuple(new_dots)

            zero = jnp.zeros((LANES,), jnp.float32)
            init = (tuple(zero for _ in range(GROUPS)),
                    tuple(zero for _ in range(GROUPS)))
            sqs, dots = lax.fori_loop(0, EMBEDDING_DIM, dim_body, init)

            for g in range(GROUPS):
                sq = jnp.maximum(sqs[g], 1e-12)
                dist = _newton_sqrt(sq)
                a = jnp.exp(dist)
                base = c * IDX_CHUNK + g * LANES
                alpha_v[pl.ds(base, LANES)] = a
                num_v[pl.ds(base, LANES)] = a * dots[g]

        for half in range(SENT_PER_W // LANES):
            sent = jnp.full((LANES,), half * LANES, jnp.int32) + lane_iota
            acc_a = jnp.zeros((LANES,), jnp.float32)
            acc_n = jnp.zeros((LANES,), jnp.float32)
            for j in range(SEQ_LEN):
                tok = sent * SEQ_LEN + j
                acc_a = acc_a + plsc.load_gather(alpha_v, [tok])
                acc_n = acc_n + plsc.load_gather(num_v, [tok])
            res_v[pl.ds(half * LANES, LANES)] = acc_n / acc_a

        pltpu.sync_copy(res_v, out_hbm.at[pl.ds(wid * SENT_PER_W, SENT_PER_W)])

    return k


_sc_kernel = _make_sc_kernel()


def kernel(batch_word_idxs, word_embeddings, weights, attend_u):
    idx = batch_word_idxs.astype(jnp.int32).reshape(NW * TOK_PER_W)
    w_flat = weights.reshape(EMBEDDING_DIM).astype(jnp.float32)
    table = jnp.pad(word_embeddings, ((0, 0), (0, PADDED_DIM - EMBEDDING_DIM)))
    out = _sc_kernel(table, idx, attend_u, w_flat)
    return out.reshape(BATCH, 1)
